# Initial kernel scaffold; baseline (speedup 1.0000x reference)
import jax
import jax.numpy as jnp
from jax.experimental import pallas as pl


def kernel(x, embed_weight):
    raise NotImplementedError("write your pallas kernel here")


# speedup vs baseline: 1.7966x; 1.7966x over previous
import functools

import jax
import jax.numpy as jnp
from jax import lax
from jax.experimental import pallas as pl
from jax.experimental.pallas import tpu as pltpu
from jax.experimental.pallas import tpu_sc as plsc

D_MODEL = 64
CHUNK = 512


@functools.lru_cache(maxsize=None)
def _make_gather(vocab: int, n_rows: int, d: int):
    info = plsc.get_sparse_core_info()
    num_cores, num_subcores = info.num_cores, info.num_subcores
    n_workers = num_cores * num_subcores
    per_worker = n_rows // n_workers
    assert n_rows % n_workers == 0
    assert per_worker % CHUNK == 0
    n_chunks = per_worker // CHUNK
    mesh = plsc.VectorSubcoreMesh(core_axis_name="c", subcore_axis_name="s")

    @functools.partial(
        pl.kernel,
        mesh=mesh,
        out_type=jax.ShapeDtypeStruct((n_rows, d), jnp.float32),
        scratch_types=[
            pltpu.VMEM((CHUNK,), jnp.int32),
            pltpu.VMEM((CHUNK, d), jnp.float32),
            pltpu.SemaphoreType.DMA,
        ],
        compiler_params=pltpu.CompilerParams(use_tc_tiling_on_sc=False),
    )
    def gather_kernel(table_hbm, idx_hbm, out_hbm, idx_v, rows_v, sem):
        wid = lax.axis_index("s") * num_cores + lax.axis_index("c")
        base = wid * per_worker

        def body(i, carry):
            off = base + i * CHUNK
            pltpu.sync_copy(idx_hbm.at[pl.ds(off, CHUNK)], idx_v)
            pltpu.async_copy(table_hbm.at[idx_v], rows_v, sem).wait()
            pltpu.sync_copy(rows_v, out_hbm.at[pl.ds(off, CHUNK)])
            return carry

        lax.fori_loop(0, n_chunks, body, 0)

    return gather_kernel


def kernel(x, embed_weight):
    batch, hist = x.shape
    vocab, d = embed_weight.shape
    idx = x.reshape(batch * hist).astype(jnp.int32)
    out = _make_gather(vocab, batch * hist, d)(embed_weight, idx)
    return out.reshape(batch, hist, d)


# speedup vs baseline: 2.5104x; 1.3973x over previous
import functools

import jax
import jax.numpy as jnp
from jax import lax
from jax.experimental import pallas as pl
from jax.experimental.pallas import tpu as pltpu
from jax.experimental.pallas import tpu_sc as plsc

HIST = 50
BATCH = 16384
D_MODEL = 64
VOCAB = 1000000


@functools.lru_cache(maxsize=None)
def _make_colgather():
    info = plsc.get_sparse_core_info()
    num_cores, num_subcores = info.num_cores, info.num_subcores
    e_per_core = D_MODEL // num_cores
    b_per_tile = BATCH // num_subcores
    mesh = plsc.VectorSubcoreMesh(core_axis_name="c", subcore_axis_name="s")

    @functools.partial(
        pl.kernel,
        mesh=mesh,
        out_type=jax.ShapeDtypeStruct((HIST, D_MODEL, BATCH), jnp.float32),
        scratch_types=[
            pltpu.VMEM((HIST * b_per_tile,), jnp.int32),
            pltpu.VMEM((b_per_tile,), jnp.float32),
            pltpu.VMEM((b_per_tile,), jnp.float32),
            pltpu.VMEM_SHARED((VOCAB,), jnp.float32),
            pltpu.SemaphoreType.DMA,
            pltpu.SemaphoreType.DMA,
            pltpu.SemaphoreType.DMA,
        ],
    )
    def colgather(wt_hbm, xt_hbm, out_hbm, idx_v, dst_a, dst_b, row_sp,
                  gsem, wsem_a, wsem_b):
        cid = lax.axis_index("c")
        sid = lax.axis_index("s")
        b0 = sid * b_per_tile

        def load_idx(h, carry):
            pltpu.sync_copy(xt_hbm.at[h, pl.ds(b0, b_per_tile)],
                            idx_v.at[pl.ds(h * b_per_tile, b_per_tile)])
            return carry

        lax.fori_loop(0, HIST, load_idx, 0)

        def e_body(ei, carry):
            e = cid * e_per_core + ei
            @pl.when(sid == 0)
            def _():
                pltpu.sync_copy(wt_hbm.at[e, :], row_sp)

            plsc.subcore_barrier()

            def h_body(j, carry2):
                h = 2 * j
                pltpu.async_copy(
                    row_sp.at[idx_v.at[pl.ds(h * b_per_tile, b_per_tile)]],
                    dst_a, gsem).wait()
                copy_a = pltpu.make_async_copy(
                    dst_a, out_hbm.at[h, e, pl.ds(b0, b_per_tile)], wsem_a)
                copy_a.start()
                pltpu.async_copy(
                    row_sp.at[idx_v.at[pl.ds((h + 1) * b_per_tile,
                                             b_per_tile)]],
                    dst_b, gsem).wait()
                copy_b = pltpu.make_async_copy(
                    dst_b, out_hbm.at[h + 1, e, pl.ds(b0, b_per_tile)], wsem_b)
                copy_b.start()
                copy_a.wait()
                copy_b.wait()
                return carry2

            lax.fori_loop(0, HIST // 2, h_body, 0)
            plsc.subcore_barrier()
            return carry

        lax.fori_loop(0, e_per_core, e_body, 0)

    return colgather


def kernel(x, embed_weight):
    xt = x.T.astype(jnp.int32)
    wt = embed_weight.T
    out3 = _make_colgather()(wt, xt)
    return jnp.transpose(out3, (2, 0, 1))


# speedup vs baseline: 3.2001x; 1.2747x over previous
import functools

import jax
import jax.numpy as jnp
from jax import lax
from jax.experimental import pallas as pl
from jax.experimental.pallas import tpu as pltpu
from jax.experimental.pallas import tpu_sc as plsc

HIST = 50
BATCH = 16384
D_MODEL = 64
VOCAB = 1000000
HB = 5


@functools.lru_cache(maxsize=None)
def _make_colgather():
    info = plsc.get_sparse_core_info()
    num_cores, num_subcores = info.num_cores, info.num_subcores
    e_per_core = D_MODEL // num_cores
    b_per_tile = BATCH // num_subcores
    hb = HB * b_per_tile
    mesh = plsc.VectorSubcoreMesh(core_axis_name="c", subcore_axis_name="s")

    @functools.partial(
        pl.kernel,
        mesh=mesh,
        out_type=jax.ShapeDtypeStruct((HIST, D_MODEL, BATCH), jnp.float32),
        scratch_types=[
            pltpu.VMEM((HIST * b_per_tile,), jnp.int32),
            pltpu.VMEM((hb,), jnp.float32),
            pltpu.VMEM((hb,), jnp.float32),
            pltpu.VMEM_SHARED((VOCAB,), jnp.float32),
            pltpu.SemaphoreType.DMA,
            pltpu.SemaphoreType.DMA,
            pltpu.SemaphoreType.DMA,
        ],
    )
    def colgather(wt_hbm, xt_hbm, out_hbm, idx_v, dst_a, dst_b, sp,
                  gsem, wsem_a, wsem_b):
        cid = lax.axis_index("c")
        sid = lax.axis_index("s")
        b0 = sid * b_per_tile
        e_base = cid * e_per_core

        def load_idx(h, carry):
            pltpu.sync_copy(xt_hbm.at[h, pl.ds(b0, b_per_tile)],
                            idx_v.at[pl.ds(h * b_per_tile, b_per_tile)])
            return carry

        lax.fori_loop(0, HIST, load_idx, 0)

        dummy_hb = wt_hbm.at[0, pl.ds(0, hb)]

        def half_batch(e, sp, h_off, dst, wsem, not_first):
            if not_first is None:
                pltpu.make_async_copy(dst, out_hbm.at[0, 0, pl.ds(0, hb)],
                                      wsem).wait()
            else:
                @pl.when(not_first)
                def _():
                    pltpu.make_async_copy(dst, out_hbm.at[0, 0, pl.ds(0, hb)],
                                          wsem).wait()

            def fire_gather(j, carry):
                h = h_off + j
                pltpu.make_async_copy(
                    sp.at[idx_v.at[pl.ds(h * b_per_tile, b_per_tile)]],
                    dst.at[pl.ds(j * b_per_tile, b_per_tile)],
                    gsem).start()
                return carry

            lax.fori_loop(0, HB, fire_gather, 0)
            pltpu.make_async_copy(dummy_hb, dst, gsem).wait()

            def fire_write(j, carry):
                h = h_off + j
                pltpu.make_async_copy(
                    dst.at[pl.ds(j * b_per_tile, b_per_tile)],
                    out_hbm.at[h, e, pl.ds(b0, b_per_tile)],
                    wsem).start()
                return carry

            lax.fori_loop(0, HB, fire_write, 0)

        def do_e(i, carry):
            e = e_base + i

            @pl.when(sid == 0)
            def _():
                pltpu.sync_copy(wt_hbm.at[e, :], sp)

            plsc.subcore_barrier()

            def t_body(t, c):
                nf = jnp.logical_or(i > 0, t > 0)
                half_batch(e, sp, (2 * t) * HB, dst_a, wsem_a, nf)
                half_batch(e, sp, (2 * t + 1) * HB, dst_b, wsem_b, nf)
                return c

            lax.fori_loop(0, HIST // (2 * HB), t_body, 0)
            plsc.subcore_barrier()
            return carry

        lax.fori_loop(0, e_per_core, do_e, 0)

        pltpu.make_async_copy(dst_a, out_hbm.at[0, 0, pl.ds(0, hb)],
                              wsem_a).wait()
        pltpu.make_async_copy(dst_b, out_hbm.at[0, 0, pl.ds(0, hb)],
                              wsem_b).wait()

    return colgather


def kernel(x, embed_weight):
    xt = x.T.astype(jnp.int32)
    wt = embed_weight.T
    out3 = _make_colgather()(wt, xt)
    return jnp.transpose(out3, (2, 0, 1))


# speedup vs baseline: 3.2610x; 1.0190x over previous
import functools

import jax
import jax.numpy as jnp
from jax import lax
from jax.experimental import pallas as pl
from jax.experimental.pallas import tpu as pltpu
from jax.experimental.pallas import tpu_sc as plsc

HIST = 50
BATCH = 16384
D_MODEL = 64
VOCAB = 1000000
HB = 5


@functools.lru_cache(maxsize=None)
def _make_colgather():
    info = plsc.get_sparse_core_info()
    num_cores, num_subcores = info.num_cores, info.num_subcores
    e_per_core = D_MODEL // num_cores
    b_per_tile = BATCH // num_subcores
    hb = HB * b_per_tile
    mesh = plsc.VectorSubcoreMesh(core_axis_name="c", subcore_axis_name="s")

    @functools.partial(
        pl.kernel,
        mesh=mesh,
        out_type=jax.ShapeDtypeStruct((HIST, D_MODEL, BATCH), jnp.float32),
        scratch_types=[
            pltpu.VMEM((HIST * b_per_tile,), jnp.int32),
            pltpu.VMEM((hb,), jnp.float32),
            pltpu.VMEM((hb,), jnp.float32),
            pltpu.VMEM_SHARED((VOCAB,), jnp.float32),
            pltpu.SemaphoreType.DMA,
            pltpu.SemaphoreType.DMA,
            pltpu.SemaphoreType.DMA,
            pltpu.SemaphoreType.DMA,
        ],
    )
    def colgather(wt_hbm, xt_hbm, out_hbm, idx_v, dst_a, dst_b, sp,
                  gsem_a, gsem_b, wsem_a, wsem_b):
        cid = lax.axis_index("c")
        sid = lax.axis_index("s")
        b0 = sid * b_per_tile
        e_base = cid * e_per_core

        def load_idx(h, carry):
            pltpu.sync_copy(xt_hbm.at[h, pl.ds(b0, b_per_tile)],
                            idx_v.at[pl.ds(h * b_per_tile, b_per_tile)])
            return carry

        lax.fori_loop(0, HIST, load_idx, 0)

        def reclaim(dst, wsem, guard):
            if guard is None:
                pltpu.make_async_copy(dst, out_hbm.at[0, 0, pl.ds(0, hb)],
                                      wsem).wait()
            else:
                @pl.when(guard)
                def _():
                    pltpu.make_async_copy(dst, out_hbm.at[0, 0, pl.ds(0, hb)],
                                          wsem).wait()

        def fire_gather(m, dst, gsem):
            pltpu.make_async_copy(
                sp.at[idx_v.at[pl.ds(m * hb, hb)]], dst, gsem).start()

        def drain_gather(dst, gsem):
            pltpu.make_async_copy(wt_hbm.at[0, pl.ds(0, hb)], dst,
                                  gsem).wait()

        def fire_writes(m, e, dst, wsem):
            def body(j, carry):
                h = m * HB + j
                pltpu.make_async_copy(
                    dst.at[pl.ds(j * b_per_tile, b_per_tile)],
                    out_hbm.at[h, e, pl.ds(b0, b_per_tile)],
                    wsem).start()
                return carry

            lax.fori_loop(0, HB, body, 0)

        def do_e(i, carry):
            e = e_base + i

            @pl.when(sid == 0)
            def _():
                pltpu.sync_copy(wt_hbm.at[e, :], sp)

            plsc.subcore_barrier()

            reclaim(dst_a, wsem_a, i > 0)
            fire_gather(0, dst_a, gsem_a)

            def t_body(t, c):
                reclaim(dst_b, wsem_b, jnp.logical_or(i > 0, t > 0))
                fire_gather(2 * t + 1, dst_b, gsem_b)
                drain_gather(dst_a, gsem_a)
                fire_writes(2 * t, e, dst_a, wsem_a)
                reclaim(dst_a, wsem_a, None)
                fire_gather(2 * t + 2, dst_a, gsem_a)
                drain_gather(dst_b, gsem_b)
                fire_writes(2 * t + 1, e, dst_b, wsem_b)
                return c

            n_batch = HIST // HB
            lax.fori_loop(0, n_batch // 2 - 1, t_body, 0)
            reclaim(dst_b, wsem_b, None)
            fire_gather(n_batch - 1, dst_b, gsem_b)
            drain_gather(dst_a, gsem_a)
            fire_writes(n_batch - 2, e, dst_a, wsem_a)
            drain_gather(dst_b, gsem_b)
            fire_writes(n_batch - 1, e, dst_b, wsem_b)
            plsc.subcore_barrier()
            return carry

        lax.fori_loop(0, e_per_core, do_e, 0)

        pltpu.make_async_copy(dst_a, out_hbm.at[0, 0, pl.ds(0, hb)],
                              wsem_a).wait()
        pltpu.make_async_copy(dst_b, out_hbm.at[0, 0, pl.ds(0, hb)],
                              wsem_b).wait()

    return colgather


def kernel(x, embed_weight):
    xt = x.T.astype(jnp.int32)
    wt = embed_weight.T
    out3 = _make_colgather()(wt, xt)
    return jnp.transpose(out3, (2, 0, 1))


# speedup vs baseline: 3.2777x; 1.0051x over previous
import functools

import jax
import jax.numpy as jnp
from jax import lax
from jax.experimental import pallas as pl
from jax.experimental.pallas import tpu as pltpu
from jax.experimental.pallas import tpu_sc as plsc

HIST = 50
BATCH = 16384
D_MODEL = 64
VOCAB = 1000000


@functools.lru_cache(maxsize=None)
def _make_colgather():
    info = plsc.get_sparse_core_info()
    num_cores, num_subcores = info.num_cores, info.num_subcores
    e_per_core = D_MODEL // num_cores
    b_per_tile = BATCH // num_subcores
    mesh = plsc.VectorSubcoreMesh(core_axis_name="c", subcore_axis_name="s")

    @functools.partial(
        pl.kernel,
        mesh=mesh,
        out_type=jax.ShapeDtypeStruct((HIST, D_MODEL, BATCH), jnp.float32),
        scratch_types=[
            [pltpu.VMEM((b_per_tile,), jnp.int32) for _ in range(3)],
            [pltpu.VMEM((b_per_tile,), jnp.float32) for _ in range(2)],
            pltpu.VMEM_SHARED((VOCAB,), jnp.float32),
            pltpu.VMEM_SHARED((VOCAB,), jnp.float32),
            [pltpu.SemaphoreType.DMA for _ in range(3)],
            [pltpu.SemaphoreType.DMA for _ in range(2)],
            [pltpu.SemaphoreType.DMA for _ in range(2)],
            pltpu.SemaphoreType.DMA,
            pltpu.SemaphoreType.DMA,
        ],
    )
    def colgather(wt_hbm, xt_hbm, out_hbm, idx_ring, dsts, sp_a, sp_b,
                  isems, gsems, wsems, ssem_a, ssem_b):
        cid = lax.axis_index("c")
        sid = lax.axis_index("s")
        b0 = sid * b_per_tile
        e_base = cid * e_per_core

        dummy_i32 = xt_hbm.at[0, pl.ds(0, b_per_tile)]
        dummy_f32 = wt_hbm.at[0, pl.ds(0, b_per_tile)]

        def wait_idx(r):
            pltpu.make_async_copy(dummy_i32, idx_ring[r], isems[r]).wait()

        def fire_idxload(m, r):
            pltpu.make_async_copy(xt_hbm.at[m, pl.ds(b0, b_per_tile)],
                                  idx_ring[r], isems[r]).start()

        def reclaim(x, guard):
            def w():
                pltpu.make_async_copy(
                    dsts[x], out_hbm.at[0, 0, pl.ds(0, b_per_tile)],
                    wsems[x]).wait()
            if guard is None:
                w()
            else:
                pl.when(guard)(w)

        def fire_gather(sp, r, x):
            pltpu.make_async_copy(sp.at[idx_ring[r]], dsts[x],
                                  gsems[x]).start()

        def finish_batch(m_prev, e, y):
            pltpu.make_async_copy(dummy_f32, dsts[y], gsems[y]).wait()
            pltpu.make_async_copy(
                dsts[y], out_hbm.at[m_prev, e, pl.ds(b0, b_per_tile)],
                wsems[y]).start()

        @pl.when(sid == 0)
        def _():
            pltpu.make_async_copy(wt_hbm.at[e_base, :], sp_a, ssem_a).start()
            pltpu.make_async_copy(wt_hbm.at[e_base + 1, :], sp_b,
                                  ssem_b).start()

        def do_e(i, e, sp, ssem, k):
            fire_idxload(0, 0)
            fire_idxload(1, 1)
            fire_idxload(2, 2)

            @pl.when(sid == 0)
            def _():
                pltpu.make_async_copy(wt_hbm.at[e, :], sp, ssem).wait()

            plsc.subcore_barrier()

            def t_loop(t, c):
                for j in range(6):
                    m = 6 * t + j
                    r = j % 3
                    x = j % 2
                    guard = jnp.logical_or(i > 0, t > 0) if j < 2 else None
                    wait_idx(r)
                    reclaim(x, guard)
                    fire_gather(sp, r, x)
                    r2 = (j + 2) % 3
                    if j == 0:
                        @pl.when(t > 0)
                        def _(m=m, r2=r2):
                            finish_batch(m - 1, e, 1)
                            fire_idxload(m + 2, r2)
                    else:
                        finish_batch(m - 1, e, 1 - x)

                        @pl.when(m + 2 < HIST)
                        def _(m=m, r2=r2):
                            fire_idxload(m + 2, r2)
                return c

            lax.fori_loop(0, HIST // 6, t_loop, 0)

            for m, r, x in ((48, 0, 0), (49, 1, 1)):
                wait_idx(r)
                reclaim(x, None)
                fire_gather(sp, r, x)
                finish_batch(m - 1, e, 1 - x)
            finish_batch(HIST - 1, e, 1)

            plsc.subcore_barrier()

            @pl.when(jnp.logical_and(sid == 0, k < e_per_core // 2 - 1))
            def _():
                pltpu.make_async_copy(wt_hbm.at[e + 2, :], sp, ssem).start()

        def e_pair(k, carry):
            do_e(2 * k, e_base + 2 * k, sp_a, ssem_a, k)
            do_e(2 * k + 1, e_base + 2 * k + 1, sp_b, ssem_b, k)
            return carry

        lax.fori_loop(0, e_per_core // 2, e_pair, 0)

        reclaim(0, None)
        reclaim(1, None)

    return colgather


def kernel(x, embed_weight):
    xt = x.T.astype(jnp.int32)
    wt = embed_weight.T
    out3 = _make_colgather()(wt, xt)
    return jnp.transpose(out3, (2, 0, 1))
